# baseline (speedup 1.0000x reference)
import functools

import jax
import jax.numpy as jnp
from jax import lax
from jax.experimental import pallas as pl
from jax.experimental.pallas import tpu as pltpu
from jax.experimental.pallas import tpu_sc as plsc

NUM_ROWS = 1000000
DIM = 64
BATCH = 4096
SEQ = 200
HALF0 = 128
HALF1 = SEQ - HALF0
NC, NS = 2, 16
NW = NC * NS
ROWS_PER_W = BATCH // NW
SLAB = 2 * ROWS_PER_W
NBUF = 8
G = 6
S = NBUF - G
GROUP = 16
N_GROUPS = SLAB // GROUP

_mesh = plsc.VectorSubcoreMesh(core_axis_name="c", subcore_axis_name="s")


def _split_body(x_ref, o_ref):
    x = x_ref[...]
    first = x[:, :HALF0]
    tail = x[:, HALF0:]
    tail = jnp.concatenate(
        [tail, jnp.zeros((8, HALF0 - HALF1), jnp.int32)], axis=1
    )
    o_ref[...] = jnp.concatenate([first, tail], axis=0)


_split = pl.pallas_call(
    _split_body,
    grid=(BATCH // 8,),
    in_specs=[pl.BlockSpec((8, SEQ), lambda i: (i, 0))],
    out_specs=pl.BlockSpec((16, HALF0), lambda i: (i, 0)),
    out_shape=jax.ShapeDtypeStruct((2 * BATCH, HALF0), jnp.int32),
)


@functools.partial(
    pl.kernel,
    mesh=_mesh,
    out_type=jax.ShapeDtypeStruct((BATCH, SEQ, DIM), jnp.float32),
    scratch_types=[
        pltpu.VMEM((SLAB, HALF0), jnp.int32),
        pltpu.VMEM((NBUF, HALF0, DIM), jnp.float32),
        pltpu.SemaphoreType.DMA,
        pltpu.SemaphoreType.DMA,
    ],
    compiler_params=pltpu.CompilerParams(use_tc_tiling_on_sc=False),
)
def _gather_kernel(idx_hbm, table_hbm, out_hbm, idx_v, rows_v, gsem, ssem):
    wid = lax.axis_index("s") * NC + lax.axis_index("c")
    base = wid * ROWS_PER_W
    pltpu.sync_copy(idx_hbm.at[pl.ds(wid * SLAB, SLAB)], idx_v)

    def gather(row, buf):
        pltpu.async_copy(table_hbm.at[idx_v.at[row]], rows_v.at[buf], gsem)

    def store(g, b, buf):
        brow = base + 8 * g + (b % 8)
        if b < 8:
            pltpu.async_copy(
                rows_v.at[buf], out_hbm.at[brow, pl.ds(0, HALF0)], ssem
            )
        else:
            pltpu.async_copy(
                rows_v.at[buf, pl.ds(0, HALF1)],
                out_hbm.at[brow, pl.ds(HALF0, HALF1)],
                ssem,
            )

    def wait_gather(buf):
        pltpu.make_async_copy(
            out_hbm.at[0, pl.ds(0, HALF0)], rows_v.at[buf], gsem
        ).wait()

    def wait_store(b):
        full = (2 <= b <= 9)
        if full:
            pltpu.make_async_copy(
                rows_v.at[0], out_hbm.at[0, pl.ds(0, HALF0)], ssem
            ).wait()
        else:
            pltpu.make_async_copy(
                rows_v.at[0, pl.ds(0, HALF1)],
                out_hbm.at[0, pl.ds(HALF0, HALF1)],
                ssem,
            ).wait()

    for b in range(G):
        gather(b, b)

    for b in range(GROUP):
        wait_gather(b % NBUF)
        store(0, b, b % NBUF)
        if b >= S:
            wait_store(b)
        gather(b + G, (b + G) % NBUF)

    def group(g, carry):
        for b in range(GROUP):
            wait_gather(b % NBUF)
            store(g, b, b % NBUF)
            wait_store(b)
            gather(g * GROUP + b + G, (b + G) % NBUF)
        return carry

    lax.fori_loop(1, N_GROUPS - 1, group, 0)

    g = N_GROUPS - 1
    for b in range(GROUP):
        wait_gather(b % NBUF)
        store(g, b, b % NBUF)
        wait_store(b)
        if b + G < GROUP:
            gather(g * GROUP + b + G, (b + G) % NBUF)

    for b in range(GROUP, GROUP + S):
        wait_store(b)


def kernel(input_ids, weight):
    idx2 = _split(input_ids.astype(jnp.int32))
    return _gather_kernel(idx2, weight)


# speedup vs baseline: 4.7261x; 4.7261x over previous
import functools

import jax
import jax.numpy as jnp
from jax import lax
from jax.experimental import pallas as pl
from jax.experimental.pallas import tpu as pltpu
from jax.experimental.pallas import tpu_sc as plsc

NUM_ROWS = 1000000
DIM = 64
PDIM = 128
BATCH = 4096
SEQ = 200
NC, NS = 2, 16
NW = NC * NS
ROWS_PER_W = BATCH // NW
NBUF = 4
G = 2
S = NBUF - G
N_GROUPS = ROWS_PER_W // NBUF

_mesh = plsc.VectorSubcoreMesh(core_axis_name="c", subcore_axis_name="s")


@functools.partial(
    pl.kernel,
    mesh=_mesh,
    out_type=jax.ShapeDtypeStruct((BATCH, SEQ, DIM), jnp.float32),
    scratch_types=[
        pltpu.VMEM((ROWS_PER_W, SEQ), jnp.int32),
        pltpu.VMEM((NBUF, SEQ, PDIM), jnp.float32),
        pltpu.SemaphoreType.DMA,
        pltpu.SemaphoreType.DMA,
    ],
    compiler_params=pltpu.CompilerParams(use_tc_tiling_on_sc=False),
)
def _gather_kernel(idx_hbm, table_hbm, out_hbm, idx_v, rows_v, gsem, ssem):
    wid = lax.axis_index("s") * NC + lax.axis_index("c")
    base = wid * ROWS_PER_W
    pltpu.sync_copy(idx_hbm.at[pl.ds(base, ROWS_PER_W)], idx_v)

    def gather(row, buf):
        pltpu.async_copy(table_hbm.at[idx_v.at[row]], rows_v.at[buf], gsem)

    def store(row, buf):
        pltpu.async_copy(
            rows_v.at[buf, :, pl.ds(0, DIM)], out_hbm.at[base + row], ssem
        )

    def wait_gather(buf):
        pltpu.make_async_copy(
            table_hbm.at[pl.ds(0, SEQ)], rows_v.at[buf], gsem
        ).wait()

    def wait_store(buf):
        pltpu.make_async_copy(
            rows_v.at[buf, :, pl.ds(0, DIM)], out_hbm.at[base], ssem
        ).wait()

    for b in range(G):
        gather(b, b)

    for b in range(NBUF):
        wait_gather(b)
        store(b, b)
        if b >= S:
            wait_store(b)
        gather(b + G, (b + G) % NBUF)

    def group(g, carry):
        j0 = g * NBUF
        for b in range(NBUF):
            j = j0 + b
            wait_gather(b)
            store(j, b)
            wait_store(b)
            gather(j + G, (b + G) % NBUF)
        return carry

    lax.fori_loop(1, N_GROUPS - 1, group, 0)

    j0 = (N_GROUPS - 1) * NBUF
    for b in range(NBUF):
        j = j0 + b
        wait_gather(b)
        store(j, b)
        wait_store(b)
        if j + G < ROWS_PER_W:
            gather(j + G, (b + G) % NBUF)

    for b in range(S):
        wait_store(b)


def kernel(input_ids, weight):
    wpad = jnp.pad(weight, ((0, 0), (0, PDIM - DIM)))
    return _gather_kernel(input_ids.astype(jnp.int32), wpad)


# speedup vs baseline: 4.7913x; 1.0138x over previous
import functools

import jax
import jax.numpy as jnp
from jax import lax
from jax.experimental import pallas as pl
from jax.experimental.pallas import tpu as pltpu
from jax.experimental.pallas import tpu_sc as plsc

NUM_ROWS = 1000000
DIM = 64
BATCH = 4096
SEQ = 200
NC, NS = 2, 16
NW = NC * NS
ROWS_PER_W = BATCH // NW
NBUF = 8
G = 6
S = NBUF - G
N_GROUPS = ROWS_PER_W // NBUF

_mesh = plsc.VectorSubcoreMesh(core_axis_name="c", subcore_axis_name="s")


@functools.partial(
    pl.kernel,
    mesh=_mesh,
    out_type=jax.ShapeDtypeStruct((BATCH, SEQ, DIM), jnp.float32),
    scratch_types=[
        pltpu.VMEM((ROWS_PER_W, SEQ), jnp.int32),
        pltpu.VMEM((NBUF, SEQ, DIM), jnp.float32),
        pltpu.SemaphoreType.DMA,
        pltpu.SemaphoreType.DMA,
    ],
    compiler_params=pltpu.CompilerParams(use_tc_tiling_on_sc=False),
)
def _gather_kernel(idx_hbm, table_hbm, out_hbm, idx_v, rows_v, gsem, ssem):
    wid = lax.axis_index("s") * NC + lax.axis_index("c")
    base = wid * ROWS_PER_W
    pltpu.sync_copy(idx_hbm.at[pl.ds(base, ROWS_PER_W)], idx_v)

    def gather(row, buf):
        pltpu.async_copy(table_hbm.at[idx_v.at[row]], rows_v.at[buf], gsem)

    def store(row, buf):
        pltpu.async_copy(rows_v.at[buf], out_hbm.at[base + row], ssem)

    def wait_gather(buf):
        pltpu.make_async_copy(out_hbm.at[base], rows_v.at[buf], gsem).wait()

    def wait_store(buf):
        pltpu.make_async_copy(rows_v.at[buf], out_hbm.at[base], ssem).wait()

    for b in range(G):
        gather(b, b)

    for b in range(NBUF):
        wait_gather(b)
        store(b, b)
        if b >= S:
            wait_store(b)
        gather(b + G, (b + G) % NBUF)

    def group(g, carry):
        j0 = g * NBUF
        for b in range(NBUF):
            j = j0 + b
            wait_gather(b)
            store(j, b)
            wait_store(b)
            gather(j + G, (b + G) % NBUF)
        return carry

    lax.fori_loop(1, N_GROUPS - 1, group, 0)

    j0 = (N_GROUPS - 1) * NBUF
    for b in range(NBUF):
        j = j0 + b
        wait_gather(b)
        store(j, b)
        wait_store(b)
        if j + G < ROWS_PER_W:
            gather(j + G, (b + G) % NBUF)

    for b in range(S):
        wait_store(b)


def kernel(input_ids, weight):
    return _gather_kernel(input_ids.astype(jnp.int32), weight)


# speedup vs baseline: 4.8010x; 1.0020x over previous
import functools

import jax
import jax.numpy as jnp
from jax import lax
from jax.experimental import pallas as pl
from jax.experimental.pallas import tpu as pltpu
from jax.experimental.pallas import tpu_sc as plsc

NUM_ROWS = 1000000
DIM = 64
BATCH = 4096
SEQ = 200
NC, NS = 2, 16
NW = NC * NS
ROWS_PER_W = BATCH // NW
R = 4
NG = ROWS_PER_W // R

_mesh = plsc.VectorSubcoreMesh(core_axis_name="c", subcore_axis_name="s")


@functools.partial(
    pl.kernel,
    mesh=_mesh,
    out_type=jax.ShapeDtypeStruct((BATCH, SEQ, DIM), jnp.float32),
    scratch_types=[
        pltpu.VMEM((ROWS_PER_W * SEQ,), jnp.int32),
        pltpu.VMEM((2, R * SEQ, DIM), jnp.float32),
        pltpu.SemaphoreType.DMA,
        pltpu.SemaphoreType.DMA,
    ],
    compiler_params=pltpu.CompilerParams(use_tc_tiling_on_sc=False),
)
def _gather_kernel(idx_hbm, table_hbm, out_hbm, idx_v, rows_v, gsem, ssem):
    wid = lax.axis_index("s") * NC + lax.axis_index("c")
    base = wid * ROWS_PER_W
    pltpu.sync_copy(idx_hbm.at[pl.ds(base * SEQ, ROWS_PER_W * SEQ)], idx_v)

    def gather(g, buf):
        pltpu.async_copy(
            table_hbm.at[idx_v.at[pl.ds(g * R * SEQ, R * SEQ)]],
            rows_v.at[buf],
            gsem,
        )

    def store(g, buf):
        for r in range(R):
            pltpu.async_copy(
                rows_v.at[buf, pl.ds(r * SEQ, SEQ)],
                out_hbm.at[base + g * R + r],
                ssem,
            )

    def wait_gather(buf):
        pltpu.make_async_copy(
            table_hbm.at[pl.ds(0, R * SEQ)], rows_v.at[buf], gsem
        ).wait()

    def wait_store(buf):
        for r in range(R):
            pltpu.make_async_copy(
                rows_v.at[buf, pl.ds(r * SEQ, SEQ)], out_hbm.at[base], ssem
            ).wait()

    def step(g, b):
        wait_gather(b)
        store(g, b)
        wait_store(1 - b)
        gather(g + 1, 1 - b)

    gather(0, 0)
    gather(1, 1)
    wait_gather(0)
    store(0, 0)

    def body(i, carry):
        step(2 * i + 1, 1)
        step(2 * i + 2, 0)
        return carry

    lax.fori_loop(0, (NG - 2) // 2, body, 0)

    wait_gather(1)
    store(NG - 1, 1)
    wait_store(0)
    wait_store(1)


def kernel(input_ids, weight):
    flat_ids = input_ids.astype(jnp.int32).reshape(-1)
    return _gather_kernel(flat_ids, weight)
